# baseline (speedup 1.0000x reference)
import functools

import jax
import jax.numpy as jnp
from jax.experimental import pallas as pl
from jax.experimental.pallas import tpu as pltpu


def _prep_kernel(x_ref, xb_ref, xtb_ref):
    xb = x_ref[...].astype(jnp.bfloat16)
    xb_ref[...] = xb
    xtb_ref[...] = xb.T


def _masked_dots(e_num, xb, lhs_bf16, lhs_sum):
    parts = []
    for e in range(e_num - 1):
        m = jnp.where(xb == e, jnp.bfloat16(1), jnp.bfloat16(0))
        parts.append(jnp.dot(lhs_bf16, m, preferred_element_type=jnp.float32))
    total = parts[0]
    for p in parts[1:]:
        total = total + p
    last = lhs_sum - total
    return jnp.concatenate(parts + [last], axis=0)


def _fused_kernel(nblk, block, e_num, xtb_ref, xb_ref, at0_ref, ut0_ref,
                  w2_ref, w1_ref, at_out_ref, ut_out_ref, at_s, ut_s):
    p = pl.program_id(0)
    b = pl.program_id(1)

    @pl.when((p == 0) & (b == 0))
    def _init():
        at_s[...] = at0_ref[...]
        ut_s[...] = ut0_ref[...]

    @pl.when((p == 0) | (p == 2))
    def _worker_pass():
        u = ut_s[...].astype(jnp.bfloat16)
        usum = jnp.sum(u.astype(jnp.float32), axis=1, keepdims=True)
        s = _masked_dots(e_num, xtb_ref[...], u, usum)
        m_a = jnp.dot(w2_ref[...], s, preferred_element_type=jnp.float32)
        sl = pl.ds(b * block, block)
        at_s[:, sl] = at_s[:, sl] + m_a

    @pl.when((p == 1) | (p == 3))
    def _task_pass():
        a = at_s[...].astype(jnp.bfloat16)
        asum = jnp.sum(a.astype(jnp.float32), axis=1, keepdims=True)
        st = _masked_dots(e_num, xb_ref[...], a, asum)
        m_t = jnp.dot(w1_ref[...], st, preferred_element_type=jnp.float32)
        sl = pl.ds(b * block, block)
        z = ut_s[:, sl] + m_t
        z = z - jnp.max(z, axis=0, keepdims=True)
        q = jnp.exp(z)
        unew = q / jnp.sum(q, axis=0, keepdims=True)
        ut_s[:, sl] = unew

        @pl.when(p == 3)
        def _emit_t():
            ut_out_ref[...] = unew

    @pl.when((p == 3) & (b == nblk - 1))
    def _emit_a():
        at_out_ref[...] = at_s[...]


def kernel(first_a, first_t, padding_a, padding_t, Awij, Awij2, inputs):
    e_num, a_num, _ = Awij.shape
    w_num, t_num = inputs.shape
    block = 1024
    nblk = w_num // block

    prep_block = 512
    xb, xtb = pl.pallas_call(
        _prep_kernel,
        grid=(w_num // prep_block,),
        in_specs=[pl.BlockSpec((prep_block, t_num), lambda i: (i, 0))],
        out_specs=[
            pl.BlockSpec((prep_block, t_num), lambda i: (i, 0)),
            pl.BlockSpec((t_num, prep_block), lambda i: (0, i)),
        ],
        out_shape=[
            jax.ShapeDtypeStruct((w_num, t_num), jnp.bfloat16),
            jax.ShapeDtypeStruct((t_num, w_num), jnp.bfloat16),
        ],
    )(inputs)
    at = first_a.T
    ut = first_t.T
    w2r = jnp.transpose(Awij2, (2, 0, 1)).reshape(a_num, e_num * e_num)
    w1r = jnp.transpose(Awij, (2, 0, 1)).reshape(e_num, e_num * a_num)

    def _a_phase(p, b):
        return ((p == 0) | (p == 2)).astype(jnp.int32)

    at_new, ut_new = pl.pallas_call(
        functools.partial(_fused_kernel, nblk, block, e_num),
        grid=(4, nblk),
        in_specs=[
            pl.BlockSpec((t_num, block), lambda p, b: (0, b * _a_phase(p, b))),
            pl.BlockSpec((w_num, block),
                         lambda p, b: (0, b * (1 - _a_phase(p, b)))),
            pl.BlockSpec((a_num, w_num), lambda p, b: (0, 0)),
            pl.BlockSpec((e_num, t_num), lambda p, b: (0, 0)),
            pl.BlockSpec(w2r.shape, lambda p, b: (0, 0)),
            pl.BlockSpec(w1r.shape, lambda p, b: (0, 0)),
        ],
        out_specs=[
            pl.BlockSpec((a_num, w_num), lambda p, b: (0, 0)),
            pl.BlockSpec((e_num, block),
                         lambda p, b: (0, b * (p == 3).astype(jnp.int32))),
        ],
        out_shape=[
            jax.ShapeDtypeStruct((a_num, w_num), jnp.float32),
            jax.ShapeDtypeStruct((e_num, t_num), jnp.float32),
        ],
        scratch_shapes=[
            pltpu.VMEM((a_num, w_num), jnp.float32),
            pltpu.VMEM((e_num, t_num), jnp.float32),
        ],
    )(xtb, xb, at, ut, w2r, w1r)

    top = jnp.concatenate([at_new.T, padding_a], axis=1)
    bot = jnp.concatenate([ut_new.T, padding_t], axis=1)
    return jnp.concatenate([top, bot], axis=0)


# speedup vs baseline: 1.0286x; 1.0286x over previous
import functools

import jax
import jax.numpy as jnp
from jax.experimental import pallas as pl
from jax.experimental.pallas import tpu as pltpu


def _prep_kernel(x_ref, xb_ref, xtb_ref):
    xb = x_ref[...].astype(jnp.bfloat16)
    xb_ref[...] = xb
    xtb_ref[...] = xb.T


def _masked_dots(e_num, xb, lhs_bf16, lhs_sum):
    parts = []
    for e in range(e_num - 1):
        m = jnp.where(xb == e, jnp.bfloat16(1), jnp.bfloat16(0))
        parts.append(jnp.dot(lhs_bf16, m, preferred_element_type=jnp.float32))
    total = parts[0]
    for p in parts[1:]:
        total = total + p
    last = lhs_sum - total
    return jnp.concatenate(parts + [last], axis=0)


def _fused_kernel(nblk, block, e_num, xtb_ref, xb_ref, at0_ref, ut0_ref,
                  w2_ref, w1_ref, at_out_ref, ut_out_ref, at_s, ut_s):
    p = pl.program_id(0)
    b = pl.program_id(1)

    @pl.when((p == 0) & (b == 0))
    def _init():
        at_s[...] = at0_ref[...]
        ut_s[...] = ut0_ref[...]

    @pl.when((p == 0) | (p == 2))
    def _worker_pass():
        u = ut_s[...].astype(jnp.bfloat16)
        usum = jnp.sum(u.astype(jnp.float32), axis=1, keepdims=True)
        s = _masked_dots(e_num, xtb_ref[...], u, usum)
        m_a = jnp.dot(w2_ref[...], s, preferred_element_type=jnp.float32)
        sl = pl.ds(b * block, block)
        at_s[:, sl] = at_s[:, sl] + m_a

    @pl.when((p == 1) | (p == 3))
    def _task_pass():
        a = at_s[...].astype(jnp.bfloat16)
        asum = jnp.sum(a.astype(jnp.float32), axis=1, keepdims=True)
        st = _masked_dots(e_num, xb_ref[...], a, asum)
        m_t = jnp.dot(w1_ref[...], st, preferred_element_type=jnp.float32)
        sl = pl.ds(b * block, block)
        z = ut_s[:, sl] + m_t
        z = z - jnp.max(z, axis=0, keepdims=True)
        q = jnp.exp(z)
        unew = q / jnp.sum(q, axis=0, keepdims=True)
        ut_s[:, sl] = unew

        @pl.when(p == 3)
        def _emit_t():
            ut_out_ref[...] = unew

    @pl.when((p == 3) & (b == nblk - 1))
    def _emit_a():
        at_out_ref[...] = at_s[...]


def kernel(first_a, first_t, padding_a, padding_t, Awij, Awij2, inputs):
    e_num, a_num, _ = Awij.shape
    w_num, t_num = inputs.shape
    block = 256
    nblk = w_num // block

    prep_block = 256
    xb, xtb = pl.pallas_call(
        _prep_kernel,
        grid=(w_num // prep_block,),
        in_specs=[pl.BlockSpec((prep_block, t_num), lambda i: (i, 0))],
        out_specs=[
            pl.BlockSpec((prep_block, t_num), lambda i: (i, 0)),
            pl.BlockSpec((t_num, prep_block), lambda i: (0, i)),
        ],
        out_shape=[
            jax.ShapeDtypeStruct((w_num, t_num), jnp.bfloat16),
            jax.ShapeDtypeStruct((t_num, w_num), jnp.bfloat16),
        ],
    )(inputs)
    at = first_a.T
    ut = first_t.T
    w2r = jnp.transpose(Awij2, (2, 0, 1)).reshape(a_num, e_num * e_num)
    w1r = jnp.transpose(Awij, (2, 0, 1)).reshape(e_num, e_num * a_num)

    def _a_phase(p, b):
        return ((p == 0) | (p == 2)).astype(jnp.int32)

    at_new, ut_new = pl.pallas_call(
        functools.partial(_fused_kernel, nblk, block, e_num),
        grid=(4, nblk),
        in_specs=[
            pl.BlockSpec((t_num, block), lambda p, b: (0, b * _a_phase(p, b))),
            pl.BlockSpec((w_num, block),
                         lambda p, b: (0, b * (1 - _a_phase(p, b)))),
            pl.BlockSpec((a_num, w_num), lambda p, b: (0, 0)),
            pl.BlockSpec((e_num, t_num), lambda p, b: (0, 0)),
            pl.BlockSpec(w2r.shape, lambda p, b: (0, 0)),
            pl.BlockSpec(w1r.shape, lambda p, b: (0, 0)),
        ],
        out_specs=[
            pl.BlockSpec((a_num, w_num), lambda p, b: (0, 0)),
            pl.BlockSpec((e_num, block),
                         lambda p, b: (0, b * (p == 3).astype(jnp.int32))),
        ],
        out_shape=[
            jax.ShapeDtypeStruct((a_num, w_num), jnp.float32),
            jax.ShapeDtypeStruct((e_num, t_num), jnp.float32),
        ],
        scratch_shapes=[
            pltpu.VMEM((a_num, w_num), jnp.float32),
            pltpu.VMEM((e_num, t_num), jnp.float32),
        ],
    )(xtb, xb, at, ut, w2r, w1r)

    top = jnp.concatenate([at_new.T, padding_a], axis=1)
    bot = jnp.concatenate([ut_new.T, padding_t], axis=1)
    return jnp.concatenate([top, bot], axis=0)


# speedup vs baseline: 1.1016x; 1.0711x over previous
import functools

import jax
import jax.numpy as jnp
from jax.experimental import pallas as pl
from jax.experimental.pallas import tpu as pltpu


def _masked_dots(e_num, xb, lhs_bf16, lhs_sum):
    parts = []
    for e in range(e_num - 1):
        m = jnp.where(xb == e, jnp.bfloat16(1), jnp.bfloat16(0))
        parts.append(jnp.dot(lhs_bf16, m, preferred_element_type=jnp.float32))
    total = parts[0]
    for p in parts[1:]:
        total = total + p
    last = lhs_sum - total
    return jnp.concatenate(parts + [last], axis=0)


def _fused_kernel(nblk, block, e_num, a_num, xa_ref, xb_ref, fa_ref, ft_ref,
                  pa_ref, pt_ref, w2_ref, w1_ref, out_ref, at_s, ut_s):
    p = pl.program_id(0)
    b = pl.program_id(1)
    w_num = at_s.shape[1]
    t_num = ut_s.shape[1]

    @pl.when((p == 0) & (b == 0))
    def _init():
        at_s[...] = fa_ref[...].T
        ut_s[...] = ft_ref[...].T

    @pl.when((p == 0) | (p == 2))
    def _worker_pass():
        xt = xa_ref[...].astype(jnp.bfloat16).T
        u = ut_s[...].astype(jnp.bfloat16)
        usum = jnp.sum(u.astype(jnp.float32), axis=1, keepdims=True)
        s = _masked_dots(e_num, xt, u, usum)
        m_a = jnp.dot(w2_ref[...], s, preferred_element_type=jnp.float32)
        sl = pl.ds(b * block, block)
        at_s[:, sl] = at_s[:, sl] + m_a

    @pl.when((p == 1) | (p == 3))
    def _task_pass():
        xn = xb_ref[...].astype(jnp.bfloat16)
        a = at_s[...].astype(jnp.bfloat16)
        asum = jnp.sum(a.astype(jnp.float32), axis=1, keepdims=True)
        st = _masked_dots(e_num, xn, a, asum)
        m_t = jnp.dot(w1_ref[...], st, preferred_element_type=jnp.float32)
        sl = pl.ds(b * block, block)
        z = ut_s[:, sl] + m_t
        z = z - jnp.max(z, axis=0, keepdims=True)
        q = jnp.exp(z)
        ut_s[:, sl] = q / jnp.sum(q, axis=0, keepdims=True)

    @pl.when((p == 3) & (b == nblk - 1))
    def _emit():
        out_ref[0:w_num, 0:a_num] = at_s[...].T
        out_ref[0:w_num, a_num:] = pa_ref[...]
        out_ref[w_num:, 0:e_num] = ut_s[...].T
        out_ref[w_num:, e_num:] = pt_ref[...]


def kernel(first_a, first_t, padding_a, padding_t, Awij, Awij2, inputs):
    e_num, a_num, _ = Awij.shape
    w_num, t_num = inputs.shape
    block = 512
    nblk = w_num // block

    w2r = jnp.transpose(Awij2, (2, 0, 1)).reshape(a_num, e_num * e_num)
    w1r = jnp.transpose(Awij, (2, 0, 1)).reshape(e_num, e_num * a_num)

    def _is_a(p):
        return (p == 0) | (p == 2)

    def _xa_idx(p, b):
        return (jnp.where(_is_a(p), b, nblk - 1), 0)

    def _xb_idx(p, b):
        inactive = jnp.where(p == 0, 0, nblk - 1)
        return (0, jnp.where(_is_a(p), inactive, b))

    out = pl.pallas_call(
        functools.partial(_fused_kernel, nblk, block, e_num, a_num),
        grid=(4, nblk),
        in_specs=[
            pl.BlockSpec((block, t_num), _xa_idx),
            pl.BlockSpec((w_num, block), _xb_idx),
            pl.BlockSpec((w_num, a_num), lambda p, b: (0, 0)),
            pl.BlockSpec((t_num, e_num), lambda p, b: (0, 0)),
            pl.BlockSpec((w_num, e_num), lambda p, b: (0, 0)),
            pl.BlockSpec((t_num, a_num), lambda p, b: (0, 0)),
            pl.BlockSpec(w2r.shape, lambda p, b: (0, 0)),
            pl.BlockSpec(w1r.shape, lambda p, b: (0, 0)),
        ],
        out_specs=pl.BlockSpec((w_num + t_num, a_num + e_num),
                               lambda p, b: (0, 0)),
        out_shape=jax.ShapeDtypeStruct((w_num + t_num, a_num + e_num),
                                       jnp.float32),
        scratch_shapes=[
            pltpu.VMEM((a_num, w_num), jnp.float32),
            pltpu.VMEM((e_num, t_num), jnp.float32),
        ],
    )(inputs, inputs, first_a, first_t, padding_a, padding_t, w2r, w1r)
    return out
